# baseline (speedup 1.0000x reference)
import functools

import jax
import jax.numpy as jnp
import numpy as np
from jax.experimental import pallas as pl

H = 128


def _silu(x):
    return x * jax.nn.sigmoid(x)


def _pre_body(h_ref, w_ref, cond_ref, wf_ref, bf_ref, pre1_ref, pre2_ref, film_ref):
    t = h_ref[...] @ w_ref[...]
    pre1_ref[...] = t[:, :H]
    pre2_ref[...] = t[:, H:]
    film_ref[...] = cond_ref[...] @ wf_ref[...] + bf_ref[...]


def _pre_call(h, w1ab, cond, wf, bf, cn):
    n = h.shape[0]
    b_cond, cd = cond.shape
    grid = (n // cn,)
    return pl.pallas_call(
        _pre_body,
        grid=grid,
        in_specs=[
            pl.BlockSpec((cn, H), lambda b: (b, 0)),
            pl.BlockSpec((H, 2 * H), lambda b: (0, 0)),
            pl.BlockSpec((b_cond, cd), lambda b: (0, 0)),
            pl.BlockSpec((cd, 2 * H), lambda b: (0, 0)),
            pl.BlockSpec((1, 2 * H), lambda b: (0, 0)),
        ],
        out_specs=[
            pl.BlockSpec((cn, H), lambda b: (b, 0)),
            pl.BlockSpec((cn, H), lambda b: (b, 0)),
            pl.BlockSpec((b_cond, 2 * H), lambda b: (0, 0)),
        ],
        out_shape=[
            jax.ShapeDtypeStruct((n, H), jnp.float32),
            jax.ShapeDtypeStruct((n, H), jnp.float32),
            jax.ShapeDtypeStruct((b_cond, 2 * H), jnp.float32),
        ],
    )(h, w1ab, cond, wf, bf)


def _edge_body(g1_ref, g2_ref, geo_ref, ea_ref, w1d_ref, w1c_ref, b1_ref,
               w2_ref, b2_ref, wc1_ref, bc1_ref, wc2_ref, m2_ref, cu_ref):
    geo = geo_ref[...]
    dist = geo[:, 3:4]
    t = (g1_ref[...] + g2_ref[...] + dist * w1c_ref[...]
         + ea_ref[...] @ w1d_ref[...] + b1_ref[...])
    m = _silu(t)
    m2 = _silu(m @ w2_ref[...] + b2_ref[...])
    q = _silu(m2 @ wc1_ref[...] + bc1_ref[...])
    cw = q @ wc2_ref[...]
    s = cw * jax.lax.rsqrt(dist + 1e-8)
    mask = (jax.lax.broadcasted_iota(jnp.int32, (1, 8), 1) < 3).astype(jnp.float32)
    cu_ref[...] = geo * s * mask
    m2_ref[...] = m2


def _edge_call(g1, g2, geo, ea, w1d, w1c, b1, w2, b2, wc1, bc1, wc2, ce):
    e = g1.shape[0]
    ed = ea.shape[1]
    grid = (e // ce,)
    const = lambda b: (0, 0)
    return pl.pallas_call(
        _edge_body,
        grid=grid,
        in_specs=[
            pl.BlockSpec((ce, H), lambda b: (b, 0)),
            pl.BlockSpec((ce, H), lambda b: (b, 0)),
            pl.BlockSpec((ce, 8), lambda b: (b, 0)),
            pl.BlockSpec((ce, ed), lambda b: (b, 0)),
            pl.BlockSpec((ed, H), const),
            pl.BlockSpec((1, H), const),
            pl.BlockSpec((1, H), const),
            pl.BlockSpec((H, H), const),
            pl.BlockSpec((1, H), const),
            pl.BlockSpec((H, H), const),
            pl.BlockSpec((1, H), const),
            pl.BlockSpec((H, 1), const),
        ],
        out_specs=[
            pl.BlockSpec((ce, H), lambda b: (b, 0)),
            pl.BlockSpec((ce, 8), lambda b: (b, 0)),
        ],
        out_shape=[
            jax.ShapeDtypeStruct((e, H), jnp.float32),
            jax.ShapeDtypeStruct((e, 8), jnp.float32),
        ],
    )(g1, g2, geo, ea, w1d, w1c, b1, w2, b2, wc1, bc1, wc2)


def _node_body(h_ref, a0_ref, a1_ref, film_ref, posp_ref, c0_ref, c1_ref,
               wn1a_ref, wn1b_ref, bn1_ref, wn2_ref, bn2_ref, lng_ref, lnb_ref,
               hout_ref, posout_ref):
    h = h_ref[...]
    agg = a0_ref[...] + a1_ref[...]
    u = _silu(h @ wn1a_ref[...] + agg @ wn1b_ref[...] + bn1_ref[...])
    v = u @ wn2_ref[...] + bn2_ref[...]
    film = film_ref[...]
    v = film[:, :H] * v + film[:, H:]
    r = h + v
    mu = jnp.mean(r, axis=-1, keepdims=True)
    d = r - mu
    var = jnp.mean(d * d, axis=-1, keepdims=True)
    hout_ref[...] = d * jax.lax.rsqrt(var + 1e-5) * lng_ref[...] + lnb_ref[...]
    posout_ref[...] = posp_ref[...] + c0_ref[...] + c1_ref[...]


def _node_call(h, a0, a1, film, posp, c0, c1, wn1a, wn1b, bn1, wn2, bn2,
               lng, lnb, cn):
    n = h.shape[0]
    grid = (n // cn,)
    const = lambda b: (0, 0)
    blk = lambda w: pl.BlockSpec((cn, w), lambda b: (b, 0))
    return pl.pallas_call(
        _node_body,
        grid=grid,
        in_specs=[
            blk(H), blk(H), blk(H), blk(2 * H), blk(8), blk(8), blk(8),
            pl.BlockSpec((H, H), const),
            pl.BlockSpec((H, H), const),
            pl.BlockSpec((1, H), const),
            pl.BlockSpec((H, H), const),
            pl.BlockSpec((1, H), const),
            pl.BlockSpec((1, H), const),
            pl.BlockSpec((1, H), const),
        ],
        out_specs=[blk(H), blk(8)],
        out_shape=[
            jax.ShapeDtypeStruct((n, H), jnp.float32),
            jax.ShapeDtypeStruct((n, 8), jnp.float32),
        ],
    )(h, a0, a1, film, posp, c0, c1, wn1a, wn1b, bn1, wn2, bn2, lng, lnb)


def kernel(h, pos, edge_attr, cond, W1, b1, W2, b2, Wc1, bc1, Wc2, Wn1, bn1,
           Wn2, bn2, Wf, bf, ln_g, ln_b, edge_index, batch):
    n, _ = h.shape
    e = edge_attr.shape[0]
    i = edge_index[0].astype(jnp.int32)
    j = edge_index[1].astype(jnp.int32)

    w1ab = jnp.concatenate([W1[:H], W1[H:2 * H]], axis=1)
    w1c = W1[2 * H:2 * H + 1]
    w1d = W1[2 * H + 1:]
    wn1a = Wn1[:H]
    wn1b = Wn1[H:]
    row = lambda v: v.reshape(1, -1)

    pre1, pre2, film_params = _pre_call(h, w1ab, cond, Wf, row(bf), 1000)

    g1 = pre1[i]
    g2 = pre2[j]
    rel = pos[i] - pos[j]
    dist = jnp.sum(rel * rel, axis=-1, keepdims=True)
    geo = jnp.concatenate([rel, dist, jnp.zeros((e, 4), jnp.float32)], axis=1)

    m2, cu = _edge_call(g1, g2, geo, edge_attr, w1d, w1c, row(b1), W2, row(b2),
                        Wc1, row(bc1), Wc2, 512)

    a0 = jnp.zeros((n, H), jnp.float32).at[i].add(m2)
    a1 = jnp.zeros((n, H), jnp.float32)
    c0 = jnp.zeros((n, 8), jnp.float32).at[i].add(cu)
    c1 = jnp.zeros((n, 8), jnp.float32)
    film = film_params[batch]

    posp = jnp.pad(pos, ((0, 0), (0, 5)))
    h_new, pos_new = _node_call(h, a0, a1, film, posp, c0, c1, wn1a, wn1b,
                                row(bn1), Wn2, row(bn2), row(ln_g), row(ln_b),
                                1000)
    return (h_new, pos_new[:, :3])


# speedup vs baseline: 3.4023x; 3.4023x over previous
import functools

import jax
import jax.numpy as jnp
from jax import lax
import numpy as np
from jax.experimental import pallas as pl
from jax.experimental.pallas import tpu as pltpu
from jax.experimental.pallas import tpu_sc as plsc

H = 128
NW = 32
CH = 80
PW = 16


def _silu(x):
    return x * jax.nn.sigmoid(x)


def _pre_body(h_ref, w_ref, cond_ref, wf_ref, bf_ref, pre1_ref, pre2_ref, film_ref):
    t = h_ref[...] @ w_ref[...]
    pre1_ref[...] = t[:, :H]
    pre2_ref[...] = t[:, H:]
    film_ref[...] = cond_ref[...] @ wf_ref[...] + bf_ref[...]


def _pre_call(h, w1ab, cond, wf, bf, cn):
    n = h.shape[0]
    b_cond, cd = cond.shape
    return pl.pallas_call(
        _pre_body,
        grid=(n // cn,),
        in_specs=[
            pl.BlockSpec((cn, H), lambda b: (b, 0)),
            pl.BlockSpec((H, 2 * H), lambda b: (0, 0)),
            pl.BlockSpec((b_cond, cd), lambda b: (0, 0)),
            pl.BlockSpec((cd, 2 * H), lambda b: (0, 0)),
            pl.BlockSpec((1, 2 * H), lambda b: (0, 0)),
        ],
        out_specs=[
            pl.BlockSpec((cn, H), lambda b: (b, 0)),
            pl.BlockSpec((cn, H), lambda b: (b, 0)),
            pl.BlockSpec((b_cond, 2 * H), lambda b: (0, 0)),
        ],
        out_shape=[
            jax.ShapeDtypeStruct((n, H), jnp.float32),
            jax.ShapeDtypeStruct((n, H), jnp.float32),
            jax.ShapeDtypeStruct((b_cond, 2 * H), jnp.float32),
        ],
    )(h, w1ab, cond, wf, bf)


def _geo_call(posf, i3, j3):
    nch = i3.shape[1]
    e = NW * nch * CH
    pwe = nch * CH
    nf = posf.shape[0]
    mesh = plsc.VectorSubcoreMesh(core_axis_name="c", subcore_axis_name="s")

    def body(posf_hbm, i3_hbm, j3_hbm, geo_hbm, posv, iv, jv, geov):
        c = lax.axis_index("c")
        s = lax.axis_index("s")
        w = s * 2 + c
        pltpu.sync_copy(posf_hbm, posv)
        pltpu.sync_copy(i3_hbm.at[w], iv)
        pltpu.sync_copy(j3_hbm.at[w], jv)
        lanes = lax.iota(jnp.int32, 16)

        def step(q, carry):
            for t in range(CH // 16):
                ii = iv[q, pl.ds(t * 16, 16)]
                jj = jv[q, pl.ds(t * 16, 16)]
                fi = ii * 4
                fj = jj * 4
                rx = plsc.load_gather(posv, [fi]) - plsc.load_gather(posv, [fj])
                ry = plsc.load_gather(posv, [fi + 1]) - plsc.load_gather(posv, [fj + 1])
                rz = plsc.load_gather(posv, [fi + 2]) - plsc.load_gather(posv, [fj + 2])
                d = rx * rx + ry * ry + rz * rz
                rows = lanes + t * 16
                plsc.store_scatter(geov, [rows, jnp.full((16,), 0, jnp.int32)], rx)
                plsc.store_scatter(geov, [rows, jnp.full((16,), 1, jnp.int32)], ry)
                plsc.store_scatter(geov, [rows, jnp.full((16,), 2, jnp.int32)], rz)
                plsc.store_scatter(geov, [rows, jnp.full((16,), 3, jnp.int32)], d)
            pltpu.sync_copy(geov, geo_hbm.at[pl.ds(w * pwe + q * CH, CH)])
            return carry

        lax.fori_loop(0, nch, step, 0)

    f = pl.kernel(
        body,
        out_type=jax.ShapeDtypeStruct((e, 8), jnp.float32),
        mesh=mesh,
        compiler_params=pltpu.CompilerParams(needs_layout_passes=False),
        scratch_types=[
            pltpu.VMEM((nf,), jnp.float32),
            pltpu.VMEM((nch, CH), jnp.int32),
            pltpu.VMEM((nch, CH), jnp.int32),
            pltpu.VMEM((CH, 8), jnp.float32),
        ],
    )
    return f(posf, i3, j3)


def _gather_call(pre1, pre2, i3, j3):
    nch = i3.shape[1]
    e = NW * nch * CH
    pwe = nch * CH
    mesh = plsc.VectorSubcoreMesh(core_axis_name="c", subcore_axis_name="s")

    def body(pre1_hbm, pre2_hbm, i3_hbm, j3_hbm, g1_hbm, g2_hbm,
             iv, jv, r1, r2, s1, s2):
        c = lax.axis_index("c")
        s = lax.axis_index("s")
        w = s * 2 + c
        pltpu.sync_copy(i3_hbm.at[w], iv)
        pltpu.sync_copy(j3_hbm.at[w], jv)

        def step(q, carry):
            base = w * pwe + q * CH
            a1 = pltpu.async_copy(pre1_hbm.at[iv.at[q]], r1, s1)
            a2 = pltpu.async_copy(pre2_hbm.at[jv.at[q]], r2, s2)
            a1.wait()
            pltpu.sync_copy(r1, g1_hbm.at[pl.ds(base, CH)])
            a2.wait()
            pltpu.sync_copy(r2, g2_hbm.at[pl.ds(base, CH)])
            return carry

        lax.fori_loop(0, nch, step, 0)

    f = pl.kernel(
        body,
        out_type=[
            jax.ShapeDtypeStruct((e, H), jnp.float32),
            jax.ShapeDtypeStruct((e, H), jnp.float32),
        ],
        mesh=mesh,
        compiler_params=pltpu.CompilerParams(needs_layout_passes=False),
        scratch_types=[
            pltpu.VMEM((nch, CH), jnp.int32),
            pltpu.VMEM((nch, CH), jnp.int32),
            pltpu.VMEM((CH, H), jnp.float32),
            pltpu.VMEM((CH, H), jnp.float32),
            pltpu.SemaphoreType.DMA,
            pltpu.SemaphoreType.DMA,
        ],
    )
    return f(pre1, pre2, i3, j3)


def _scatter_call(m2, i3, z128, n16):
    nch = i3.shape[1]
    pwe = nch * CH
    zr = n16 // 16
    mesh = plsc.VectorSubcoreMesh(core_axis_name="c", subcore_axis_name="s")

    def body(m2_hbm, i3_hbm, z128_hbm, aggp_hbm, iv, rows, acc_s):
        c = lax.axis_index("c")
        s = lax.axis_index("s")
        w = s * 2 + c
        pltpu.sync_copy(z128_hbm, acc_s.at[pl.ds(s * zr, zr)])
        plsc.subcore_barrier()
        pltpu.sync_copy(i3_hbm.at[w], iv)

        def step(q, carry):
            base = w * pwe + q * CH
            pltpu.sync_copy(m2_hbm.at[pl.ds(base, CH)], rows)
            pltpu.sync_copy(rows, acc_s.at[iv.at[q]], add=True)
            return carry

        lax.fori_loop(0, nch, step, 0)
        plsc.subcore_barrier()
        pltpu.sync_copy(acc_s.at[pl.ds(s * zr, zr)],
                        aggp_hbm.at[c, pl.ds(s * zr, zr)])

    f = pl.kernel(
        body,
        out_type=jax.ShapeDtypeStruct((2, n16, H), jnp.float32),
        mesh=mesh,
        compiler_params=pltpu.CompilerParams(needs_layout_passes=False),
        scratch_types=[
            pltpu.VMEM((nch, CH), jnp.int32),
            pltpu.VMEM((CH, H), jnp.float32),
            pltpu.VMEM_SHARED((n16, H), jnp.float32),
        ],
    )
    return f(m2, i3, z128)


def _coord_call(cu, i3, film_params, batch3, z128, ident3, rc):
    nch = i3.shape[1]
    pwe = nch * CH
    nfch = batch3.shape[1]
    npad = NW * nfch * CH
    mesh = plsc.VectorSubcoreMesh(core_axis_name="c", subcore_axis_name="s")

    def body(cu_hbm, i3_hbm, film_hbm, b3_hbm, z128_hbm, id3_hbm,
             aggc_hbm, filmo_hbm,
             iv, cuv, frows, bidx, identv, accf, accc_s, sem):
        c = lax.axis_index("c")
        s = lax.axis_index("s")
        w = s * 2 + c
        pltpu.sync_copy(b3_hbm.at[w], bidx)
        for q in range(nfch):
            pltpu.async_copy(film_hbm.at[bidx.at[q]], frows, sem).wait()
            pltpu.sync_copy(frows, filmo_hbm.at[pl.ds(w * nfch * CH + q * CH, CH)])
        pltpu.sync_copy(z128_hbm.at[pl.ds(0, rc)], accf)

        @pl.when(s < rc // CH)
        def _():
            pltpu.sync_copy(z128_hbm.at[pl.ds(0, CH)],
                            accc_s.at[pl.ds(s * CH, CH)])

        plsc.subcore_barrier()
        pltpu.sync_copy(i3_hbm.at[w], iv)
        pltpu.sync_copy(id3_hbm, identv)
        lanes = lax.iota(jnp.int32, 16)

        def step(q, carry):
            base = w * pwe + q * CH
            pltpu.sync_copy(cu_hbm.at[pl.ds(base, CH)], cuv)
            for t in range(CH // 16):
                ii = iv[q, pl.ds(t * 16, 16)]
                fi = ii * 4
                erow = lanes + t * 16
                for comp in range(3):
                    vals = plsc.load_gather(
                        cuv, [erow, jnp.full((16,), comp, jnp.int32)])
                    fic = fi + comp
                    plsc.addupdate_scatter(
                        accf,
                        [lax.shift_right_logical(fic, 7),
                         lax.bitwise_and(fic, 127)],
                        vals)
            return carry

        lax.fori_loop(0, nch, step, 0)
        for q in range(rc // CH):
            pltpu.sync_copy(accf.at[pl.ds(q * CH, CH)],
                            accc_s.at[identv.at[q]], add=True)
        plsc.subcore_barrier()

        @pl.when(s < rc // CH)
        def _():
            pltpu.sync_copy(accc_s.at[pl.ds(s * CH, CH)],
                            aggc_hbm.at[c, pl.ds(s * CH, CH)])

    f = pl.kernel(
        body,
        out_type=[
            jax.ShapeDtypeStruct((2, rc, H), jnp.float32),
            jax.ShapeDtypeStruct((npad, 2 * H), jnp.float32),
        ],
        mesh=mesh,
        compiler_params=pltpu.CompilerParams(needs_layout_passes=False),
        scratch_types=[
            pltpu.VMEM((nch, CH), jnp.int32),
            pltpu.VMEM((CH, 8), jnp.float32),
            pltpu.VMEM((CH, 2 * H), jnp.float32),
            pltpu.VMEM((nfch, CH), jnp.int32),
            pltpu.VMEM((rc // CH, CH), jnp.int32),
            pltpu.VMEM((rc, H), jnp.float32),
            pltpu.VMEM_SHARED((rc, H), jnp.float32),
            pltpu.SemaphoreType.DMA,
        ],
    )
    return f(cu, i3, film_params, batch3, z128, ident3)


def _edge_body(g1_ref, g2_ref, geo_ref, ea_ref, w1d_ref, w1c_ref, b1_ref,
               w2_ref, b2_ref, wc1_ref, bc1_ref, wc2_ref, m2_ref, cu_ref):
    geo = geo_ref[...]
    dist = geo[:, 3:4]
    t = (g1_ref[...] + g2_ref[...] + dist * w1c_ref[...]
         + ea_ref[...] @ w1d_ref[...] + b1_ref[...])
    m = _silu(t)
    m2 = _silu(m @ w2_ref[...] + b2_ref[...])
    q = _silu(m2 @ wc1_ref[...] + bc1_ref[...])
    cw = q @ wc2_ref[...]
    s = cw * jax.lax.rsqrt(dist + 1e-8)
    mask = (jax.lax.broadcasted_iota(jnp.int32, (1, 8), 1) < 3).astype(jnp.float32)
    cu_ref[...] = geo * s * mask
    m2_ref[...] = m2


def _edge_call(g1, g2, geo, ea, w1d, w1c, b1, w2, b2, wc1, bc1, wc2, ce):
    e = g1.shape[0]
    ed = ea.shape[1]
    const = lambda b: (0, 0)
    return pl.pallas_call(
        _edge_body,
        grid=(e // ce,),
        in_specs=[
            pl.BlockSpec((ce, H), lambda b: (b, 0)),
            pl.BlockSpec((ce, H), lambda b: (b, 0)),
            pl.BlockSpec((ce, 8), lambda b: (b, 0)),
            pl.BlockSpec((ce, ed), lambda b: (b, 0)),
            pl.BlockSpec((ed, H), const),
            pl.BlockSpec((1, H), const),
            pl.BlockSpec((1, H), const),
            pl.BlockSpec((H, H), const),
            pl.BlockSpec((1, H), const),
            pl.BlockSpec((H, H), const),
            pl.BlockSpec((1, H), const),
            pl.BlockSpec((H, 1), const),
        ],
        out_specs=[
            pl.BlockSpec((ce, H), lambda b: (b, 0)),
            pl.BlockSpec((ce, 8), lambda b: (b, 0)),
        ],
        out_shape=[
            jax.ShapeDtypeStruct((e, H), jnp.float32),
            jax.ShapeDtypeStruct((e, 8), jnp.float32),
        ],
    )(g1, g2, geo, ea, w1d, w1c, b1, w2, b2, wc1, bc1, wc2)


def _node_body(h_ref, a0_ref, a1_ref, film_ref, posp_ref, c0_ref, c1_ref,
               wn1a_ref, wn1b_ref, bn1_ref, wn2_ref, bn2_ref, lng_ref, lnb_ref,
               hout_ref, posout_ref):
    h = h_ref[...]
    agg = a0_ref[...] + a1_ref[...]
    u = _silu(h @ wn1a_ref[...] + agg @ wn1b_ref[...] + bn1_ref[...])
    v = u @ wn2_ref[...] + bn2_ref[...]
    film = film_ref[...]
    v = film[:, :H] * v + film[:, H:]
    r = h + v
    mu = jnp.mean(r, axis=-1, keepdims=True)
    d = r - mu
    var = jnp.mean(d * d, axis=-1, keepdims=True)
    hout_ref[...] = d * jax.lax.rsqrt(var + 1e-5) * lng_ref[...] + lnb_ref[...]
    posout_ref[...] = posp_ref[...] + c0_ref[...] + c1_ref[...]


def _node_call(h, a0, a1, film, posp, c0, c1, wn1a, wn1b, bn1, wn2, bn2,
               lng, lnb, cn):
    n = h.shape[0]
    const = lambda b: (0, 0)
    blk = lambda wdt: pl.BlockSpec((cn, wdt), lambda b: (b, 0))
    return pl.pallas_call(
        _node_body,
        grid=(n // cn,),
        in_specs=[
            blk(H), blk(H), blk(H), blk(2 * H), blk(PW), blk(PW), blk(PW),
            pl.BlockSpec((H, H), const),
            pl.BlockSpec((H, H), const),
            pl.BlockSpec((1, H), const),
            pl.BlockSpec((H, H), const),
            pl.BlockSpec((1, H), const),
            pl.BlockSpec((1, H), const),
            pl.BlockSpec((1, H), const),
        ],
        out_specs=[blk(H), blk(PW)],
        out_shape=[
            jax.ShapeDtypeStruct((n, H), jnp.float32),
            jax.ShapeDtypeStruct((n, PW), jnp.float32),
        ],
    )(h, a0, a1, film, posp, c0, c1, wn1a, wn1b, bn1, wn2, bn2, lng, lnb)


def kernel(h, pos, edge_attr, cond, W1, b1, W2, b2, Wc1, bc1, Wc2, Wn1, bn1,
           Wn2, bn2, Wf, bf, ln_g, ln_b, edge_index, batch):
    n, _ = h.shape
    e = edge_attr.shape[0]
    assert e % (NW * CH) == 0 and n % 16 == 0
    nch = e // (NW * CH)
    i32 = edge_index.astype(jnp.int32)
    i3 = i32[0].reshape(NW, nch, CH)
    j3 = i32[1].reshape(NW, nch, CH)
    nfch = -(-n // (NW * CH))
    npad = NW * nfch * CH
    batch3 = jnp.pad(batch.astype(jnp.int32), (0, npad - n)).reshape(NW, nfch, CH)
    rc = -(-(n * 4) // (H * CH)) * CH
    n16 = -(-n // 128) * 128
    assert rc % 16 == 0 and n % 16 == 0 and rc // CH <= 16
    ident3 = jnp.arange(rc, dtype=jnp.int32).reshape(rc // CH, CH)

    w1ab = jnp.concatenate([W1[:H], W1[H:2 * H]], axis=1)
    w1c = W1[2 * H:2 * H + 1]
    w1d = W1[2 * H + 1:]
    wn1a = Wn1[:H]
    wn1b = Wn1[H:]
    row = lambda v: v.reshape(1, -1)
    posf = jnp.pad(pos, ((0, 0), (0, 4 - pos.shape[1]))).reshape(-1)
    posp = jnp.pad(pos, ((0, 0), (0, PW - pos.shape[1])))
    z128 = jnp.zeros((n16 // 16, H), jnp.float32)

    pre1, pre2, film_params = _pre_call(h, w1ab, cond, Wf, row(bf), 1000)
    geo = _geo_call(posf, i3, j3)
    g1, g2 = _gather_call(pre1, pre2, i3, j3)

    m2, cu = _edge_call(g1, g2, geo, edge_attr, w1d, w1c, row(b1), W2,
                        row(b2), Wc1, row(bc1), Wc2, 512)

    aggp = _scatter_call(m2, i3, z128, n16)
    aggc, film = _coord_call(cu, i3, film_params, batch3, z128, ident3, rc)

    cflat = aggc.reshape(2, rc * H)[:, :n * 4].reshape(2, n, 4)
    c0 = jnp.pad(cflat[0], ((0, 0), (0, PW - 4)))
    c1 = jnp.pad(cflat[1], ((0, 0), (0, PW - 4)))

    h_new, pos_new = _node_call(h, aggp[0, :n], aggp[1, :n], film[:n], posp,
                                c0, c1, wn1a, wn1b, row(bn1), Wn2,
                                row(bn2), row(ln_g), row(ln_b), 1000)
    return (h_new, pos_new[:, :3])


# speedup vs baseline: 3.7685x; 1.1076x over previous
import functools

import jax
import jax.numpy as jnp
from jax import lax
import numpy as np
from jax.experimental import pallas as pl
from jax.experimental.pallas import tpu as pltpu
from jax.experimental.pallas import tpu_sc as plsc

H = 128
NW = 32
CH = 80
PW = 16


def _silu(x):
    return x * jax.nn.sigmoid(x)


def _pre_body(h_ref, w_ref, cond_ref, wf_ref, bf_ref, pre1_ref, pre2_ref, film_ref):
    t = h_ref[...] @ w_ref[...]
    pre1_ref[...] = t[:, :H]
    pre2_ref[...] = t[:, H:]
    film_ref[...] = cond_ref[...] @ wf_ref[...] + bf_ref[...]


def _pre_call(h, w1ab, cond, wf, bf, cn):
    n = h.shape[0]
    b_cond, cd = cond.shape
    return pl.pallas_call(
        _pre_body,
        grid=(n // cn,),
        in_specs=[
            pl.BlockSpec((cn, H), lambda b: (b, 0)),
            pl.BlockSpec((H, 2 * H), lambda b: (0, 0)),
            pl.BlockSpec((b_cond, cd), lambda b: (0, 0)),
            pl.BlockSpec((cd, 2 * H), lambda b: (0, 0)),
            pl.BlockSpec((1, 2 * H), lambda b: (0, 0)),
        ],
        out_specs=[
            pl.BlockSpec((cn, H), lambda b: (b, 0)),
            pl.BlockSpec((cn, H), lambda b: (b, 0)),
            pl.BlockSpec((b_cond, 2 * H), lambda b: (0, 0)),
        ],
        out_shape=[
            jax.ShapeDtypeStruct((n, H), jnp.float32),
            jax.ShapeDtypeStruct((n, H), jnp.float32),
            jax.ShapeDtypeStruct((b_cond, 2 * H), jnp.float32),
        ],
    )(h, w1ab, cond, wf, bf)


def _geo_call(posf, i3, j3):
    nch = i3.shape[1]
    e = NW * nch * CH
    pwe = nch * CH
    nf = posf.shape[0]
    gpb = 25 if nch % 25 == 0 else 1
    nb = nch // gpb
    mesh = plsc.VectorSubcoreMesh(core_axis_name="c", subcore_axis_name="s")

    def body(posf_hbm, i3_hbm, j3_hbm, geo_hbm, posv, iv, jv, geov):
        c = lax.axis_index("c")
        s = lax.axis_index("s")
        w = s * 2 + c
        pltpu.sync_copy(posf_hbm, posv)
        pltpu.sync_copy(i3_hbm.at[w], iv)
        pltpu.sync_copy(j3_hbm.at[w], jv)
        lanes = lax.iota(jnp.int32, 16)

        def step(b, carry):
            for qq in range(gpb):
                for t in range(CH // 16):
                    ii = iv[b * gpb + qq, pl.ds(t * 16, 16)]
                    jj = jv[b * gpb + qq, pl.ds(t * 16, 16)]
                    fi = ii * 4
                    fj = jj * 4
                    rx = plsc.load_gather(posv, [fi]) - plsc.load_gather(posv, [fj])
                    ry = plsc.load_gather(posv, [fi + 1]) - plsc.load_gather(posv, [fj + 1])
                    rz = plsc.load_gather(posv, [fi + 2]) - plsc.load_gather(posv, [fj + 2])
                    d = rx * rx + ry * ry + rz * rz
                    r8 = (lanes + (qq * CH + t * 16)) * 8
                    plsc.store_scatter(geov, [r8], rx)
                    plsc.store_scatter(geov, [r8 + 1], ry)
                    plsc.store_scatter(geov, [r8 + 2], rz)
                    plsc.store_scatter(geov, [r8 + 3], d)
            pltpu.sync_copy(geov,
                            geo_hbm.at[pl.ds((w * pwe + b * gpb * CH) * 8,
                                             gpb * CH * 8)])
            return carry

        lax.fori_loop(0, nb, step, 0)

    f = pl.kernel(
        body,
        out_type=jax.ShapeDtypeStruct((e * 8,), jnp.float32),
        mesh=mesh,
        compiler_params=pltpu.CompilerParams(needs_layout_passes=False),
        scratch_types=[
            pltpu.VMEM((nf,), jnp.float32),
            pltpu.VMEM((nch, CH), jnp.int32),
            pltpu.VMEM((nch, CH), jnp.int32),
            pltpu.VMEM((gpb * CH * 8,), jnp.float32),
        ],
    )
    return f(posf, i3, j3)


def _gather_call(pre1, pre2, i3, j3):
    nch = i3.shape[1]
    e = NW * nch * CH
    pwe = nch * CH
    mesh = plsc.VectorSubcoreMesh(core_axis_name="c", subcore_axis_name="s")

    def body(pre1_hbm, pre2_hbm, i3_hbm, j3_hbm, g1_hbm, g2_hbm,
             iv, jv, r1, r2, s1, s2, o1, o2):
        c = lax.axis_index("c")
        s = lax.axis_index("s")
        w = s * 2 + c
        pltpu.sync_copy(i3_hbm.at[w], iv)
        pltpu.sync_copy(j3_hbm.at[w], jv)
        pltpu.async_copy(pre1_hbm.at[iv.at[0]], r1.at[0], s1)
        pltpu.async_copy(pre2_hbm.at[jv.at[0]], r2.at[0], s2)

        def step(q, carry):
            base = w * pwe + q * CH
            cur = lax.bitwise_and(q, 1)
            nxt = 1 - cur

            @pl.when(q >= 1)
            def _():
                pltpu.make_async_copy(r1.at[nxt], g1_hbm.at[pl.ds(base, CH)], o1).wait()
                pltpu.make_async_copy(r2.at[nxt], g2_hbm.at[pl.ds(base, CH)], o2).wait()

            @pl.when(q + 1 < nch)
            def _():
                pltpu.async_copy(pre1_hbm.at[iv.at[q + 1]], r1.at[nxt], s1)
                pltpu.async_copy(pre2_hbm.at[jv.at[q + 1]], r2.at[nxt], s2)

            pltpu.make_async_copy(pre1_hbm.at[iv.at[q]], r1.at[cur], s1).wait()
            pltpu.async_copy(r1.at[cur], g1_hbm.at[pl.ds(base, CH)], o1)
            pltpu.make_async_copy(pre2_hbm.at[jv.at[q]], r2.at[cur], s2).wait()
            pltpu.async_copy(r2.at[cur], g2_hbm.at[pl.ds(base, CH)], o2)
            return carry

        lax.fori_loop(0, nch, step, 0)
        last = lax.bitwise_and(nch - 1, 1)
        pltpu.make_async_copy(r1.at[last], g1_hbm.at[pl.ds(0, CH)], o1).wait()
        pltpu.make_async_copy(r2.at[last], g2_hbm.at[pl.ds(0, CH)], o2).wait()

    f = pl.kernel(
        body,
        out_type=[
            jax.ShapeDtypeStruct((e, H), jnp.float32),
            jax.ShapeDtypeStruct((e, H), jnp.float32),
        ],
        mesh=mesh,
        compiler_params=pltpu.CompilerParams(needs_layout_passes=False),
        scratch_types=[
            pltpu.VMEM((nch, CH), jnp.int32),
            pltpu.VMEM((nch, CH), jnp.int32),
            pltpu.VMEM((2, CH, H), jnp.float32),
            pltpu.VMEM((2, CH, H), jnp.float32),
            pltpu.SemaphoreType.DMA,
            pltpu.SemaphoreType.DMA,
            pltpu.SemaphoreType.DMA,
            pltpu.SemaphoreType.DMA,
        ],
    )
    return f(pre1, pre2, i3, j3)


def _scatter_call(m2, i3, z128, n16):
    nch = i3.shape[1]
    pwe = nch * CH
    zr = n16 // 16
    mesh = plsc.VectorSubcoreMesh(core_axis_name="c", subcore_axis_name="s")

    def body(m2_hbm, i3_hbm, z128_hbm, aggp_hbm, iv, rows, acc_s, sl):
        c = lax.axis_index("c")
        s = lax.axis_index("s")
        w = s * 2 + c
        pltpu.sync_copy(i3_hbm.at[w], iv)
        pltpu.sync_copy(z128_hbm, acc_s.at[pl.ds(s * zr, zr)])
        plsc.subcore_barrier()
        pltpu.async_copy(m2_hbm.at[pl.ds(w * pwe, CH)], rows.at[0], sl)

        def step(q, carry):
            cur = lax.bitwise_and(q, 1)
            nxt = 1 - cur

            @pl.when(q + 1 < nch)
            def _():
                pltpu.async_copy(m2_hbm.at[pl.ds(w * pwe + (q + 1) * CH, CH)],
                                 rows.at[nxt], sl)

            pltpu.make_async_copy(m2_hbm.at[pl.ds(w * pwe, CH)],
                                  rows.at[cur], sl).wait()
            pltpu.sync_copy(rows.at[cur], acc_s.at[iv.at[q]], add=True)
            return carry

        lax.fori_loop(0, nch, step, 0)
        plsc.subcore_barrier()
        pltpu.sync_copy(acc_s.at[pl.ds(s * zr, zr)],
                        aggp_hbm.at[c, pl.ds(s * zr, zr)])

    f = pl.kernel(
        body,
        out_type=jax.ShapeDtypeStruct((2, n16, H), jnp.float32),
        mesh=mesh,
        compiler_params=pltpu.CompilerParams(needs_layout_passes=False),
        scratch_types=[
            pltpu.VMEM((nch, CH), jnp.int32),
            pltpu.VMEM((2, CH, H), jnp.float32),
            pltpu.VMEM_SHARED((n16, H), jnp.float32),
            pltpu.SemaphoreType.DMA,
        ],
    )
    return f(m2, i3, z128)


def _coord_call(cu, i3, film_params, batch3, z128, ident3, rc):
    nch = i3.shape[1]
    pwe = nch * CH
    nfch = batch3.shape[1]
    npad = NW * nfch * CH
    gpb = 25 if nch % 25 == 0 else 1
    nb = nch // gpb
    mesh = plsc.VectorSubcoreMesh(core_axis_name="c", subcore_axis_name="s")

    def body(cu_hbm, i3_hbm, film_hbm, b3_hbm, z128_hbm, id3_hbm,
             aggc_hbm, filmo_hbm,
             iv, cuv, frows, bidx, identv, accf, accc_s, sem):
        c = lax.axis_index("c")
        s = lax.axis_index("s")
        w = s * 2 + c
        pltpu.sync_copy(b3_hbm.at[w], bidx)
        for q in range(nfch):
            pltpu.async_copy(film_hbm.at[bidx.at[q]], frows, sem).wait()
            pltpu.sync_copy(frows, filmo_hbm.at[pl.ds(w * nfch * CH + q * CH, CH)])
        pltpu.sync_copy(z128_hbm.at[pl.ds(0, rc)], accf)

        @pl.when(s < rc // CH)
        def _():
            pltpu.sync_copy(z128_hbm.at[pl.ds(0, CH)],
                            accc_s.at[pl.ds(s * CH, CH)])

        plsc.subcore_barrier()
        pltpu.sync_copy(i3_hbm.at[w], iv)
        pltpu.sync_copy(id3_hbm, identv)
        lanes = lax.iota(jnp.int32, 16)

        def step(b, carry):
            base = (w * pwe + b * gpb * CH) * 8
            pltpu.sync_copy(cu_hbm.at[pl.ds(base, gpb * CH * 8)], cuv)
            for qq in range(gpb):
                for t in range(CH // 16):
                    ii = iv[b * gpb + qq, pl.ds(t * 16, 16)]
                    fi = ii * 4
                    erow8 = (lanes + (qq * CH + t * 16)) * 8
                    for comp in range(3):
                        vals = plsc.load_gather(cuv, [erow8 + comp])
                        fic = fi + comp
                        plsc.addupdate_scatter(
                            accf,
                            [lax.shift_right_logical(fic, 7),
                             lax.bitwise_and(fic, 127)],
                            vals)
            return carry

        lax.fori_loop(0, nb, step, 0)
        for q in range(rc // CH):
            pltpu.sync_copy(accf.at[pl.ds(q * CH, CH)],
                            accc_s.at[identv.at[q]], add=True)
        plsc.subcore_barrier()

        @pl.when(s < rc // CH)
        def _():
            pltpu.sync_copy(accc_s.at[pl.ds(s * CH, CH)],
                            aggc_hbm.at[c, pl.ds(s * CH, CH)])

    f = pl.kernel(
        body,
        out_type=[
            jax.ShapeDtypeStruct((2, rc, H), jnp.float32),
            jax.ShapeDtypeStruct((npad, 2 * H), jnp.float32),
        ],
        mesh=mesh,
        compiler_params=pltpu.CompilerParams(needs_layout_passes=False),
        scratch_types=[
            pltpu.VMEM((nch, CH), jnp.int32),
            pltpu.VMEM((gpb * CH * 8,), jnp.float32),
            pltpu.VMEM((CH, 2 * H), jnp.float32),
            pltpu.VMEM((nfch, CH), jnp.int32),
            pltpu.VMEM((rc // CH, CH), jnp.int32),
            pltpu.VMEM((rc, H), jnp.float32),
            pltpu.VMEM_SHARED((rc, H), jnp.float32),
            pltpu.SemaphoreType.DMA,
        ],
    )
    return f(cu, i3, film_params, batch3, z128, ident3)


def _edge_body(g1_ref, g2_ref, geo_ref, ea_ref, w1d_ref, w1c_ref, b1_ref,
               w2_ref, b2_ref, wc1_ref, bc1_ref, wc2_ref, m2_ref, cu_ref):
    geo = geo_ref[...]
    dist = geo[:, 3:4]
    t = (g1_ref[...] + g2_ref[...] + dist * w1c_ref[...]
         + ea_ref[...] @ w1d_ref[...] + b1_ref[...])
    m = _silu(t)
    m2 = _silu(m @ w2_ref[...] + b2_ref[...])
    q = _silu(m2 @ wc1_ref[...] + bc1_ref[...])
    cw = q @ wc2_ref[...]
    s = cw * jax.lax.rsqrt(dist + 1e-8)
    mask = (jax.lax.broadcasted_iota(jnp.int32, (1, 8), 1) < 3).astype(jnp.float32)
    cu_ref[...] = geo * s * mask
    m2_ref[...] = m2


def _edge_call(g1, g2, geo, ea, w1d, w1c, b1, w2, b2, wc1, bc1, wc2, ce):
    e = g1.shape[0]
    ed = ea.shape[1]
    const = lambda b: (0, 0)
    return pl.pallas_call(
        _edge_body,
        grid=(e // ce,),
        in_specs=[
            pl.BlockSpec((ce, H), lambda b: (b, 0)),
            pl.BlockSpec((ce, H), lambda b: (b, 0)),
            pl.BlockSpec((ce, 8), lambda b: (b, 0)),
            pl.BlockSpec((ce, ed), lambda b: (b, 0)),
            pl.BlockSpec((ed, H), const),
            pl.BlockSpec((1, H), const),
            pl.BlockSpec((1, H), const),
            pl.BlockSpec((H, H), const),
            pl.BlockSpec((1, H), const),
            pl.BlockSpec((H, H), const),
            pl.BlockSpec((1, H), const),
            pl.BlockSpec((H, 1), const),
        ],
        out_specs=[
            pl.BlockSpec((ce, H), lambda b: (b, 0)),
            pl.BlockSpec((ce, 8), lambda b: (b, 0)),
        ],
        out_shape=[
            jax.ShapeDtypeStruct((e, H), jnp.float32),
            jax.ShapeDtypeStruct((e, 8), jnp.float32),
        ],
    )(g1, g2, geo, ea, w1d, w1c, b1, w2, b2, wc1, bc1, wc2)


def _node_body(h_ref, a0_ref, a1_ref, film_ref, posp_ref, c0_ref, c1_ref,
               wn1a_ref, wn1b_ref, bn1_ref, wn2_ref, bn2_ref, lng_ref, lnb_ref,
               hout_ref, posout_ref):
    h = h_ref[...]
    agg = a0_ref[...] + a1_ref[...]
    u = _silu(h @ wn1a_ref[...] + agg @ wn1b_ref[...] + bn1_ref[...])
    v = u @ wn2_ref[...] + bn2_ref[...]
    film = film_ref[...]
    v = film[:, :H] * v + film[:, H:]
    r = h + v
    mu = jnp.mean(r, axis=-1, keepdims=True)
    d = r - mu
    var = jnp.mean(d * d, axis=-1, keepdims=True)
    hout_ref[...] = d * jax.lax.rsqrt(var + 1e-5) * lng_ref[...] + lnb_ref[...]
    posout_ref[...] = posp_ref[...] + c0_ref[...] + c1_ref[...]


def _node_call(h, a0, a1, film, posp, c0, c1, wn1a, wn1b, bn1, wn2, bn2,
               lng, lnb, cn):
    n = h.shape[0]
    const = lambda b: (0, 0)
    blk = lambda wdt: pl.BlockSpec((cn, wdt), lambda b: (b, 0))
    return pl.pallas_call(
        _node_body,
        grid=(n // cn,),
        in_specs=[
            blk(H), blk(H), blk(H), blk(2 * H), blk(PW), blk(PW), blk(PW),
            pl.BlockSpec((H, H), const),
            pl.BlockSpec((H, H), const),
            pl.BlockSpec((1, H), const),
            pl.BlockSpec((H, H), const),
            pl.BlockSpec((1, H), const),
            pl.BlockSpec((1, H), const),
            pl.BlockSpec((1, H), const),
        ],
        out_specs=[blk(H), blk(PW)],
        out_shape=[
            jax.ShapeDtypeStruct((n, H), jnp.float32),
            jax.ShapeDtypeStruct((n, PW), jnp.float32),
        ],
    )(h, a0, a1, film, posp, c0, c1, wn1a, wn1b, bn1, wn2, bn2, lng, lnb)


def kernel(h, pos, edge_attr, cond, W1, b1, W2, b2, Wc1, bc1, Wc2, Wn1, bn1,
           Wn2, bn2, Wf, bf, ln_g, ln_b, edge_index, batch):
    n, _ = h.shape
    e = edge_attr.shape[0]
    assert e % (NW * CH) == 0 and n % 16 == 0
    nch = e // (NW * CH)
    i32 = edge_index.astype(jnp.int32)
    i3 = i32[0].reshape(NW, nch, CH)
    j3 = i32[1].reshape(NW, nch, CH)
    nfch = -(-n // (NW * CH))
    npad = NW * nfch * CH
    batch3 = jnp.pad(batch.astype(jnp.int32), (0, npad - n)).reshape(NW, nfch, CH)
    rc = -(-(n * 4) // (H * CH)) * CH
    n16 = -(-n // 128) * 128
    assert rc % 16 == 0 and n % 16 == 0 and rc // CH <= 16
    ident3 = jnp.arange(rc, dtype=jnp.int32).reshape(rc // CH, CH)

    w1ab = jnp.concatenate([W1[:H], W1[H:2 * H]], axis=1)
    w1c = W1[2 * H:2 * H + 1]
    w1d = W1[2 * H + 1:]
    wn1a = Wn1[:H]
    wn1b = Wn1[H:]
    row = lambda v: v.reshape(1, -1)
    posf = jnp.pad(pos, ((0, 0), (0, 4 - pos.shape[1]))).reshape(-1)
    posp = jnp.pad(pos, ((0, 0), (0, PW - pos.shape[1])))
    z128 = jnp.zeros((n16 // 16, H), jnp.float32)

    pre1, pre2, film_params = _pre_call(h, w1ab, cond, Wf, row(bf), 1000)
    geo = _geo_call(posf, i3, j3).reshape(e, 8)
    g1, g2 = _gather_call(pre1, pre2, i3, j3)

    m2, cu = _edge_call(g1, g2, geo, edge_attr, w1d, w1c, row(b1), W2,
                        row(b2), Wc1, row(bc1), Wc2, 512)

    aggp = _scatter_call(m2, i3, z128, n16)
    aggc, film = _coord_call(cu.reshape(-1), i3, film_params, batch3, z128,
                             ident3, rc)

    cflat = aggc.reshape(2, rc * H)[:, :n * 4].reshape(2, n, 4)
    c0 = jnp.pad(cflat[0], ((0, 0), (0, PW - 4)))
    c1 = jnp.pad(cflat[1], ((0, 0), (0, PW - 4)))

    h_new, pos_new = _node_call(h, aggp[0, :n], aggp[1, :n], film[:n], posp,
                                c0, c1, wn1a, wn1b, row(bn1), Wn2,
                                row(bn2), row(ln_g), row(ln_b), 1000)
    return (h_new, pos_new[:, :3])
